# baseline (speedup 1.0000x reference)
import functools

import jax
import jax.numpy as jnp
import numpy as np
from jax import lax
from jax.experimental import pallas as pl
from jax.experimental.pallas import tpu as pltpu
from jax.experimental.pallas import tpu_sc as plsc

B = 4096
NC = 13
NK = 26
V = 100000
D = 32
BH = 512
TH = 1024
C = NK + 1
CP = 32
FW = CP * D


_NW = 32
_ROWS = B * CP
_RPW = _ROWS // _NW
_CH = 128
_NCH = _RPW // _CH
_NBUF = 4


def _sc_gather(tflat, idx):
    mesh = plsc.VectorSubcoreMesh(core_axis_name="c", subcore_axis_name="s")

    @functools.partial(
        pl.kernel,
        mesh=mesh,
        compiler_params=pltpu.CompilerParams(use_tc_tiling_on_sc=False),
        out_type=jax.ShapeDtypeStruct((_ROWS, D), jnp.float32),
        scratch_types=[
            pltpu.VMEM((_RPW,), jnp.int32),
            pltpu.VMEM((_NBUF, _CH, D), jnp.float32),
            pltpu.SemaphoreType.DMA,
            pltpu.SemaphoreType.DMA,
        ],
    )
    def k(tflat_hbm, idx_hbm, out_hbm, idx_v, rows_v, gsem, osem):
        wid = lax.axis_index("s") * 2 + lax.axis_index("c")
        base = wid * _RPW
        pltpu.sync_copy(idx_hbm.at[pl.ds(base, _RPW)], idx_v)

        def group(jo):
            for b in range(_NBUF):
                ch = jo + b
                pltpu.async_copy(
                    tflat_hbm.at[idx_v.at[pl.ds(ch * _CH, _CH)]],
                    rows_v.at[b],
                    gsem,
                )
            for b in range(_NBUF):
                ch = jo + b
                pltpu.make_async_copy(
                    tflat_hbm.at[idx_v.at[pl.ds(ch * _CH, _CH)]],
                    rows_v.at[b],
                    gsem,
                ).wait()
                pltpu.async_copy(
                    rows_v.at[b],
                    out_hbm.at[pl.ds(base + ch * _CH, _CH), :],
                    osem,
                )
            for b in range(_NBUF):
                ch = jo + b
                pltpu.make_async_copy(
                    rows_v.at[b],
                    out_hbm.at[pl.ds(base + ch * _CH, _CH), :],
                    osem,
                ).wait()

        pl.loop(0, _NCH, step=_NBUF)(group)

    return k(tflat, idx)


_BB = 256


def _tc_body(cont, emb, bw1, bb1, bwh, bbh, bwo, bbo, wce, wdot, tb1,
             twh, tbh, two, tbo, out):
    f32 = jnp.float32
    x = jnp.maximum(jnp.dot(cont[...], bw1[...], preferred_element_type=f32)
                    + bb1[...], 0.0)
    x = jnp.maximum(jnp.dot(x, bwh[...], preferred_element_type=f32)
                    + bbh[...], 0.0)
    ce = jax.nn.sigmoid(jnp.dot(x, bwo[...], preferred_element_type=f32)
                        + bbo[...])

    lane = lax.broadcasted_iota(jnp.int32, (_BB, FW), 1)
    f = jnp.where(lane < NK * D, emb[...], 0.0)
    row = lax.broadcasted_iota(jnp.int32, (D, FW), 0)
    col = lax.broadcasted_iota(jnp.int32, (D, FW), 1)
    sel = (col - NK * D == row).astype(f32)
    f = f + jnp.dot(ce, sel, preferred_element_type=f32)

    f3 = f.reshape(_BB, CP, D)
    d3 = lax.dot_general(f3, f3, (((2,), (2,)), ((0,), (0,))),
                         preferred_element_type=f32)
    dv = d3.reshape(_BB, CP * CP)

    y = jnp.maximum(jnp.dot(ce, wce[...], preferred_element_type=f32)
                    + jnp.dot(dv, wdot[...], preferred_element_type=f32)
                    + tb1[...], 0.0)
    y = jnp.maximum(jnp.dot(y, twh[...], preferred_element_type=f32)
                    + tbh[...], 0.0)
    out[...] = jnp.dot(y, two[...], preferred_element_type=f32) + tbo[...]


def _tc_main(cont16, emb, bw1p, bb1, bwh, bbh, bwo, bbo, wce, wdot, tb1,
             twh, tbh, two, tbo):
    nb = B // _BB
    fixed = lambda i: (0, 0)
    return pl.pallas_call(
        _tc_body,
        grid=(nb,),
        in_specs=[
            pl.BlockSpec((_BB, 16), lambda i: (i, 0)),
            pl.BlockSpec((_BB, FW), lambda i: (i, 0)),
            pl.BlockSpec((16, BH), fixed),
            pl.BlockSpec((1, BH), fixed),
            pl.BlockSpec((BH, BH), fixed),
            pl.BlockSpec((1, BH), fixed),
            pl.BlockSpec((BH, D), fixed),
            pl.BlockSpec((1, D), fixed),
            pl.BlockSpec((D, TH), fixed),
            pl.BlockSpec((CP * CP, TH), fixed),
            pl.BlockSpec((1, TH), fixed),
            pl.BlockSpec((TH, TH), fixed),
            pl.BlockSpec((1, TH), fixed),
            pl.BlockSpec((TH, 1), fixed),
            pl.BlockSpec((1, 1), fixed),
        ],
        out_specs=pl.BlockSpec((_BB, 1), lambda i: (i, 0)),
        out_shape=jax.ShapeDtypeStruct((B, 1), jnp.float32),
    )(cont16, emb, bw1p, bb1, bwh, bbh, bwo, bbo, wce, wdot, tb1, twh,
      tbh, two, tbo)


_IU0, _IU1 = np.triu_indices(C)
_PID = np.zeros((CP, CP), np.int32)
_MSK = np.zeros((CP, CP), np.float32)
for _p, (_i, _j) in enumerate(zip(_IU0, _IU1)):
    _PID[_i, _j] = _PID[_j, _i] = _p
    _MSK[_i, _j] = _MSK[_j, _i] = 1.0 if _i == _j else 0.5
_PID_FLAT = _PID.reshape(-1)
_MSK_FLAT = _MSK.reshape(-1)[:, None]


def kernel(continuous, categorical, tables, bw1, bb1, bwh, bbh, bwo, bbo,
           tw1, tb1, twh, tbh, two, tbo):
    tflat = tables.reshape(NK * V, D)
    cat = categorical.astype(jnp.int32) + (jnp.arange(NK, dtype=jnp.int32) * V)[None, :]
    idx = jnp.pad(cat, ((0, 0), (0, CP - NK))).reshape(_ROWS)

    rows = _sc_gather(tflat, idx)
    emb = rows.reshape(B, FW)

    cont16 = jnp.pad(continuous, ((0, 0), (0, 16 - NC)))
    bw1p = jnp.pad(bw1, ((0, 16 - NC), (0, 0)))

    wce = tw1[:D, :]
    wpairs = tw1[D:, :]
    wdot = wpairs[_PID_FLAT] * _MSK_FLAT

    out = _tc_main(cont16, emb,
                   bw1p, bb1[None, :], bwh, bbh[None, :], bwo, bbo[None, :],
                   wce, wdot, tb1[None, :], twh, tbh[None, :], two,
                   tbo[None, :])
    return out


# speedup vs baseline: 1.0018x; 1.0018x over previous
import functools

import jax
import jax.numpy as jnp
import numpy as np
from jax import lax
from jax.experimental import pallas as pl
from jax.experimental.pallas import tpu as pltpu
from jax.experimental.pallas import tpu_sc as plsc

B = 4096
NC = 13
NK = 26
V = 100000
D = 32
BH = 512
TH = 1024
C = NK + 1
CP = 32
FW = CP * D


_NW = 32
_ROWS = B * CP
_RPW = _ROWS // _NW
_CH = 256
_NCH = _RPW // _CH
_NBUF = 8
_GLA = 4


def _sc_gather(tflat, idx):
    mesh = plsc.VectorSubcoreMesh(core_axis_name="c", subcore_axis_name="s")

    @functools.partial(
        pl.kernel,
        mesh=mesh,
        compiler_params=pltpu.CompilerParams(use_tc_tiling_on_sc=False),
        out_type=jax.ShapeDtypeStruct((_ROWS, D), jnp.float32),
        scratch_types=[
            pltpu.VMEM((_RPW,), jnp.int32),
            pltpu.VMEM((_NBUF, _CH, D), jnp.float32),
            pltpu.SemaphoreType.DMA((_NBUF,)),
            pltpu.SemaphoreType.DMA((_NBUF,)),
        ],
    )
    def k(tflat_hbm, idx_hbm, out_hbm, idx_v, rows_v, gsem, osem):
        wid = lax.axis_index("s") * 2 + lax.axis_index("c")
        base = wid * _RPW
        pltpu.sync_copy(idx_hbm.at[pl.ds(base, _RPW)], idx_v)

        def start_g(c):
            pltpu.async_copy(
                tflat_hbm.at[idx_v.at[pl.ds(c * _CH, _CH)]],
                rows_v.at[c % _NBUF], gsem.at[c % _NBUF])

        def wait_g(c):
            pltpu.make_async_copy(
                tflat_hbm.at[idx_v.at[pl.ds(c * _CH, _CH)]],
                rows_v.at[c % _NBUF], gsem.at[c % _NBUF]).wait()

        def start_o(c):
            pltpu.async_copy(
                rows_v.at[c % _NBUF],
                out_hbm.at[pl.ds(base + c * _CH, _CH), :], osem.at[c % _NBUF])

        def wait_o(c):
            pltpu.make_async_copy(
                rows_v.at[c % _NBUF],
                out_hbm.at[pl.ds(base + c * _CH, _CH), :],
                osem.at[c % _NBUF]).wait()

        for c in range(_NCH + _GLA):
            if c < _NCH:
                if c >= _NBUF:
                    wait_o(c - _NBUF)
                start_g(c)
            if c >= _GLA:
                wait_g(c - _GLA)
                start_o(c - _GLA)
        for c in range(_NCH - _NBUF, _NCH):
            wait_o(c)

    return k(tflat, idx)


_BB = 256


def _tc_body(cont, emb, bw1, bb1, bwh, bbh, bwo, bbo, wce, wdot, tb1,
             twh, tbh, two, tbo, out):
    f32 = jnp.float32
    x = jnp.maximum(jnp.dot(cont[...], bw1[...], preferred_element_type=f32)
                    + bb1[...], 0.0)
    x = jnp.maximum(jnp.dot(x, bwh[...], preferred_element_type=f32)
                    + bbh[...], 0.0)
    ce = jax.nn.sigmoid(jnp.dot(x, bwo[...], preferred_element_type=f32)
                        + bbo[...])

    lane = lax.broadcasted_iota(jnp.int32, (_BB, FW), 1)
    f = jnp.where(lane < NK * D, emb[...], 0.0)
    row = lax.broadcasted_iota(jnp.int32, (D, FW), 0)
    col = lax.broadcasted_iota(jnp.int32, (D, FW), 1)
    sel = (col - NK * D == row).astype(f32)
    f = f + jnp.dot(ce, sel, preferred_element_type=f32)

    f3 = f.reshape(_BB, CP, D)
    d3 = lax.dot_general(f3, f3, (((2,), (2,)), ((0,), (0,))),
                         preferred_element_type=f32)
    dv = d3.reshape(_BB, CP * CP)

    y = jnp.maximum(jnp.dot(ce, wce[...], preferred_element_type=f32)
                    + jnp.dot(dv, wdot[...], preferred_element_type=f32)
                    + tb1[...], 0.0)
    y = jnp.maximum(jnp.dot(y, twh[...], preferred_element_type=f32)
                    + tbh[...], 0.0)
    out[...] = jnp.dot(y, two[...], preferred_element_type=f32) + tbo[...]


def _tc_main(cont16, emb, bw1p, bb1, bwh, bbh, bwo, bbo, wce, wdot, tb1,
             twh, tbh, two, tbo):
    nb = B // _BB
    fixed = lambda i: (0, 0)
    return pl.pallas_call(
        _tc_body,
        grid=(nb,),
        in_specs=[
            pl.BlockSpec((_BB, 16), lambda i: (i, 0)),
            pl.BlockSpec((_BB, FW), lambda i: (i, 0)),
            pl.BlockSpec((16, BH), fixed),
            pl.BlockSpec((1, BH), fixed),
            pl.BlockSpec((BH, BH), fixed),
            pl.BlockSpec((1, BH), fixed),
            pl.BlockSpec((BH, D), fixed),
            pl.BlockSpec((1, D), fixed),
            pl.BlockSpec((D, TH), fixed),
            pl.BlockSpec((CP * CP, TH), fixed),
            pl.BlockSpec((1, TH), fixed),
            pl.BlockSpec((TH, TH), fixed),
            pl.BlockSpec((1, TH), fixed),
            pl.BlockSpec((TH, 1), fixed),
            pl.BlockSpec((1, 1), fixed),
        ],
        out_specs=pl.BlockSpec((_BB, 1), lambda i: (i, 0)),
        out_shape=jax.ShapeDtypeStruct((B, 1), jnp.float32),
    )(cont16, emb, bw1p, bb1, bwh, bbh, bwo, bbo, wce, wdot, tb1, twh,
      tbh, two, tbo)


_IU0, _IU1 = np.triu_indices(C)
_PID = np.zeros((CP, CP), np.int32)
_MSK = np.zeros((CP, CP), np.float32)
for _p, (_i, _j) in enumerate(zip(_IU0, _IU1)):
    _PID[_i, _j] = _PID[_j, _i] = _p
    _MSK[_i, _j] = _MSK[_j, _i] = 1.0 if _i == _j else 0.5
_PID_FLAT = _PID.reshape(-1)
_MSK_FLAT = _MSK.reshape(-1)[:, None]


def kernel(continuous, categorical, tables, bw1, bb1, bwh, bbh, bwo, bbo,
           tw1, tb1, twh, tbh, two, tbo):
    tflat = tables.reshape(NK * V, D)
    cat = categorical.astype(jnp.int32) + (jnp.arange(NK, dtype=jnp.int32) * V)[None, :]
    idx = jnp.pad(cat, ((0, 0), (0, CP - NK))).reshape(_ROWS)

    rows = _sc_gather(tflat, idx)
    emb = rows.reshape(B, FW)

    cont16 = jnp.pad(continuous, ((0, 0), (0, 16 - NC)))
    bw1p = jnp.pad(bw1, ((0, 16 - NC), (0, 0)))

    wce = tw1[:D, :]
    wpairs = tw1[D:, :]
    wdot = wpairs[_PID_FLAT] * _MSK_FLAT

    out = _tc_main(cont16, emb,
                   bw1p, bb1[None, :], bwh, bbh[None, :], bwo, bbo[None, :],
                   wce, wdot, tb1[None, :], twh, tbh[None, :], two,
                   tbo[None, :])
    return out
